# baseline (speedup 1.0000x reference)
import jax
import jax.numpy as jnp
from jax.experimental import pallas as pl

N_IN = 64
N_HID = 128
N_OUT = 16
BATCH = 16384


def _mlp_kernel(x_ref, w1_ref, w2_ref, bh_ref, bo_ref, rh_ref, ro_ref, o_ref):
    agg1 = jax.lax.dot_general(
        w1_ref[...], x_ref[...], (((1,), (0,)), ((), ())),
        preferred_element_type=jnp.float32,
    )
    h = jnp.tanh(bh_ref[...][:, None] + rh_ref[...][:, None] * agg1)
    agg2 = jax.lax.dot_general(
        w2_ref[...], h, (((1,), (0,)), ((), ())),
        preferred_element_type=jnp.float32,
    )
    o_ref[...] = jnp.tanh(bo_ref[...][:, None] + ro_ref[...][:, None] * agg2)


def kernel(inputs, W_ih, W_ho, b_hid, b_out, resp_hid, resp_out):
    TM = 4096
    grid = (BATCH // TM,)
    xT = inputs.T
    out_t = pl.pallas_call(
        _mlp_kernel,
        grid=grid,
        in_specs=[
            pl.BlockSpec((N_IN, TM), lambda i: (0, i)),
            pl.BlockSpec((N_HID, N_IN), lambda i: (0, 0)),
            pl.BlockSpec((N_OUT, N_HID), lambda i: (0, 0)),
            pl.BlockSpec((N_HID,), lambda i: (0,)),
            pl.BlockSpec((N_OUT,), lambda i: (0,)),
            pl.BlockSpec((N_HID,), lambda i: (0,)),
            pl.BlockSpec((N_OUT,), lambda i: (0,)),
        ],
        out_specs=pl.BlockSpec((N_OUT, TM), lambda i: (0, i)),
        out_shape=jax.ShapeDtypeStruct((N_OUT, BATCH), jnp.float32),
    )(xT, W_ih, W_ho, b_hid, b_out, resp_hid, resp_out)
    return out_t.T


# speedup vs baseline: 1.0390x; 1.0390x over previous
import jax
import jax.numpy as jnp
from jax.experimental import pallas as pl

N_IN = 64
N_HID = 128
N_OUT = 16
BATCH = 16384


def _mlp_kernel(x_ref, w1_ref, w2_ref, bh_ref, bo_ref, rh_ref, ro_ref, o_ref):
    agg1 = jax.lax.dot_general(
        w1_ref[...], x_ref[...], (((1,), (0,)), ((), ())),
        preferred_element_type=jnp.float32,
    )
    h = jnp.tanh(bh_ref[...][:, None] + rh_ref[...][:, None] * agg1)
    agg2 = jax.lax.dot_general(
        w2_ref[...], h, (((1,), (0,)), ((), ())),
        preferred_element_type=jnp.float32,
    )
    o_ref[...] = jnp.tanh(bo_ref[...][:, None] + ro_ref[...][:, None] * agg2)


def kernel(inputs, W_ih, W_ho, b_hid, b_out, resp_hid, resp_out):
    TM = 16384
    grid = (BATCH // TM,)
    xT = inputs.T
    out_t = pl.pallas_call(
        _mlp_kernel,
        grid=grid,
        in_specs=[
            pl.BlockSpec((N_IN, TM), lambda i: (0, i)),
            pl.BlockSpec((N_HID, N_IN), lambda i: (0, 0)),
            pl.BlockSpec((N_OUT, N_HID), lambda i: (0, 0)),
            pl.BlockSpec((N_HID,), lambda i: (0,)),
            pl.BlockSpec((N_OUT,), lambda i: (0,)),
            pl.BlockSpec((N_HID,), lambda i: (0,)),
            pl.BlockSpec((N_OUT,), lambda i: (0,)),
        ],
        out_specs=pl.BlockSpec((N_OUT, TM), lambda i: (0, i)),
        out_shape=jax.ShapeDtypeStruct((N_OUT, BATCH), jnp.float32),
    )(xT, W_ih, W_ho, b_hid, b_out, resp_hid, resp_out)
    return out_t.T
